# Initial kernel scaffold; baseline (speedup 1.0000x reference)
import jax
import jax.numpy as jnp
from jax.experimental import pallas as pl


def kernel(x, wave_real, wave_imag):
    raise NotImplementedError("write your pallas kernel here")


# speedup vs baseline: 1.0699x; 1.0699x over previous
import functools

import jax
import jax.numpy as jnp
from jax import lax
from jax.experimental import pallas as pl
from jax.experimental.pallas import tpu as pltpu
from jax.experimental.pallas import tpu_sc as plsc

L1, L2, ORBIT, DIM = 4, 3, 2, 2
N_SITES = L1 * L2 * ORBIT
TABLE_SIZE = DIM ** N_SITES

NC, NS, LANES = 2, 16, 16
NW = NC * NS
BATCH = 16384
B_PER_W = BATCH // NW
CHUNK = 128
N_CHUNK = B_PER_W // CHUNK
GROUPS_PER_CHUNK = CHUNK // LANES


def _wf_body(x_hbm, real_hbm, imag_hbm, out_r_hbm, out_i_hbm,
             x_v, idx_v, outr_v, outi_v, sem_r, sem_i):
    wid = lax.axis_index("s") * NC + lax.axis_index("c")
    pltpu.sync_copy(x_hbm.at[wid], x_v)

    for c in range(N_CHUNK):
        def body(g, carry, c=c):
            start = c * CHUNK + g * LANES
            acc = jnp.zeros((LANES,), jnp.int32)
            for j in range(N_SITES):
                acc = acc + acc + x_v[j, pl.ds(start, LANES)]
            idx_v[c, pl.ds(g * LANES, LANES)] = acc
            return carry
        lax.fori_loop(0, GROUPS_PER_CHUNK, body, 0)

    copies = []
    for c in range(N_CHUNK):
        copies.append(pltpu.async_copy(real_hbm.at[idx_v.at[c]], outr_v.at[c], sem_r))
        copies.append(pltpu.async_copy(imag_hbm.at[idx_v.at[c]], outi_v.at[c], sem_i))
    for cp in copies:
        cp.wait()

    pltpu.sync_copy(outr_v, out_r_hbm.at[wid])
    pltpu.sync_copy(outi_v, out_i_hbm.at[wid])


_wf = pl.kernel(
    _wf_body,
    mesh=plsc.VectorSubcoreMesh(core_axis_name="c", subcore_axis_name="s"),
    out_type=[
        jax.ShapeDtypeStruct((NW, N_CHUNK, CHUNK), jnp.float32),
        jax.ShapeDtypeStruct((NW, N_CHUNK, CHUNK), jnp.float32),
    ],
    scratch_types=[
        pltpu.VMEM((N_SITES, B_PER_W), jnp.int32),
        pltpu.VMEM((N_CHUNK, CHUNK), jnp.int32),
        pltpu.VMEM((N_CHUNK, CHUNK), jnp.float32),
        pltpu.VMEM((N_CHUNK, CHUNK), jnp.float32),
        pltpu.SemaphoreType.DMA,
        pltpu.SemaphoreType.DMA,
    ],
)


def kernel(x, wave_real, wave_imag):
    lead = x.shape[:-3]
    xf = x.reshape(lead + (N_SITES,)).reshape(-1, N_SITES)
    x3 = xf.reshape(NW, B_PER_W, N_SITES).transpose(0, 2, 1)
    real, imag = _wf(x3, wave_real, wave_imag)
    out = real.reshape(-1) + 1j * imag.reshape(-1)
    return out.reshape(lead)
